# baseline (speedup 1.0000x reference)
import functools

import numpy as np
import jax
import jax.numpy as jnp
from jax import lax
from jax.experimental import pallas as pl
from jax.experimental.pallas import tpu as pltpu
from jax.experimental.pallas import tpu_sc as plsc

_NUM_FIELDS = 26
_EMBED_DIM = 16
_BATCH = 16384
_FIELD_DIM = 100000
_OFFSETS = np.arange(_NUM_FIELDS, dtype=np.int32) * _FIELD_DIM

_NC = 2
_NS = 16
_NW = _NC * _NS
_BPW = _BATCH // _NW
_CB = 16
_ROWS = _CB * _NUM_FIELDS
_NCHUNK = _BPW // _CB
_GSUB = 104
_NG = _ROWS // _GSUB
_L = 16


def kernel(x, table):
    x_flat = x.reshape(-1)
    off = jnp.asarray(np.tile(_OFFSETS, _CB))
    mesh = plsc.VectorSubcoreMesh(core_axis_name="c", subcore_axis_name="s")

    @functools.partial(
        pl.kernel,
        mesh=mesh,
        out_type=jax.ShapeDtypeStruct((_BATCH,), jnp.float32),
        compiler_params=pltpu.CompilerParams(
            needs_layout_passes=False, use_tc_tiling_on_sc=False
        ),
        scratch_types=[
            pltpu.VMEM((_ROWS,), jnp.int32),
            pltpu.VMEM((_ROWS,), jnp.int32),
            pltpu.VMEM((_ROWS,), jnp.int32),
            pltpu.VMEM((_ROWS, _EMBED_DIM), jnp.float32),
            pltpu.VMEM((_BPW,), jnp.float32),
            pltpu.SemaphoreType.DMA,
        ],
    )
    def fm_kernel(x_hbm, off_hbm, table_hbm, out_hbm, xv, idxv, offv, rows, zbuf, sem):
        wid = lax.axis_index("s") * _NC + lax.axis_index("c")
        base_flat = wid * (_BPW * _NUM_FIELDS)
        pltpu.sync_copy(off_hbm, offv)
        lanes = lax.iota(jnp.int32, _L)

        def chunk_body(c, carry):
            flat0 = base_flat + c * _ROWS
            pltpu.sync_copy(x_hbm.at[pl.ds(flat0, _ROWS)], xv)
            for i in range(_ROWS // _L):
                sl = pl.ds(i * _L, _L)
                idxv[sl] = xv[sl] + offv[sl]
            copies = [
                pltpu.async_copy(
                    table_hbm.at[idxv.at[pl.ds(g * _GSUB, _GSUB)]],
                    rows.at[pl.ds(g * _GSUB, _GSUB), :],
                    sem,
                )
                for g in range(_NG)
            ]
            for cp in copies:
                cp.wait()
            zvec = jnp.zeros((_L,), jnp.float32)
            for b in range(_CB):
                r0 = b * _NUM_FIELDS
                v = rows[r0, :]
                s = v
                sq = v * v
                for f in range(1, _NUM_FIELDS):
                    v = rows[r0 + f, :]
                    s = s + v
                    sq = sq + v * v
                t = (s * s - sq) * 0.5
                z = jnp.sum(t)
                zvec = jnp.where(lanes == b, jnp.full((_L,), z), zvec)
            plsc.store_scatter(zbuf, [jnp.full((_L,), c * _CB) + lanes], zvec)
            return carry

        lax.fori_loop(0, _NCHUNK, chunk_body, 0)

        for i in range(_BPW // _L):
            sl = pl.ds(i * _L, _L)
            v = zbuf[sl]
            zbuf[sl] = 1.0 / (1.0 + jnp.exp(-v))
        pltpu.sync_copy(zbuf, out_hbm.at[pl.ds(wid * _BPW, _BPW)])

    return fm_kernel(x_flat, off, table)


# speedup vs baseline: 1.0262x; 1.0262x over previous
import functools

import numpy as np
import jax
import jax.numpy as jnp
from jax import lax
from jax.experimental import pallas as pl
from jax.experimental.pallas import tpu as pltpu
from jax.experimental.pallas import tpu_sc as plsc

_NUM_FIELDS = 26
_EMBED_DIM = 16
_BATCH = 16384
_FIELD_DIM = 100000
_OFFSETS = np.arange(_NUM_FIELDS, dtype=np.int32) * _FIELD_DIM

_NC = 2
_NS = 16
_NW = _NC * _NS
_BPW = _BATCH // _NW
_CB = 16
_ROWS = _CB * _NUM_FIELDS
_NCHUNK = _BPW // _CB
_GSUB = 104
_NG = _ROWS // _GSUB
_L = 16
_XLEN = _BPW * _NUM_FIELDS
_RING = 2


def kernel(x, table):
    x_flat = x.reshape(-1)
    off = jnp.asarray(np.tile(_OFFSETS, _CB))
    mesh = plsc.VectorSubcoreMesh(core_axis_name="c", subcore_axis_name="s")

    @functools.partial(
        pl.kernel,
        mesh=mesh,
        out_type=jax.ShapeDtypeStruct((_BATCH,), jnp.float32),
        compiler_params=pltpu.CompilerParams(
            needs_layout_passes=False, use_tc_tiling_on_sc=False
        ),
        scratch_types=[
            pltpu.VMEM((_XLEN,), jnp.int32),
            pltpu.VMEM((_ROWS,), jnp.int32),
            pltpu.VMEM((_ROWS, _EMBED_DIM), jnp.float32),
            pltpu.VMEM((_ROWS, _EMBED_DIM), jnp.float32),
            pltpu.VMEM((_BPW,), jnp.float32),
            pltpu.SemaphoreType.DMA,
            pltpu.SemaphoreType.DMA,
        ],
    )
    def fm_kernel(x_hbm, off_hbm, table_hbm, out_hbm,
                  idxall, offv, rows0, rows1, zbuf, sem0, sem1):
        wid = lax.axis_index("s") * _NC + lax.axis_index("c")
        base_flat = wid * _XLEN
        pltpu.sync_copy(x_hbm.at[pl.ds(base_flat, _XLEN)], idxall)
        pltpu.sync_copy(off_hbm, offv)
        lanes = lax.iota(jnp.int32, _L)
        ring = ((rows0, sem0), (rows1, sem1))

        def off_body(c, carry):
            b0 = c * _ROWS
            for i in range(_ROWS // _L):
                ii = jnp.full((_L,), b0 + i * _L) + lanes
                v = plsc.load_gather(idxall, [ii])
                plsc.store_scatter(idxall, [ii], v + offv[pl.ds(i * _L, _L)])
            return carry

        lax.fori_loop(0, _NCHUNK, off_body, 0)

        def fire(c, rows_ref, sem):
            b0 = c * _ROWS
            for g in range(_NG):
                pltpu.make_async_copy(
                    table_hbm.at[idxall.at[pl.ds(b0 + g * _GSUB, _GSUB)]],
                    rows_ref.at[pl.ds(g * _GSUB, _GSUB), :],
                    sem,
                ).start()

        def drain(rows_ref, sem):
            pltpu.make_async_copy(
                table_hbm.at[pl.ds(0, _ROWS), :], rows_ref, sem
            ).wait()

        fire(0, rows0, sem0)
        fire(1, rows1, sem1)

        def round_body(g, carry):
            for r in range(_RING):
                c = g * _RING + r
                rows_ref, sem = ring[r]
                drain(rows_ref, sem)
                zvec = jnp.zeros((_L,), jnp.float32)
                for b in range(_CB):
                    r0 = b * _NUM_FIELDS
                    v = rows_ref[r0, :]
                    s = v
                    sq = v * v
                    for f in range(1, _NUM_FIELDS):
                        v = rows_ref[r0 + f, :]
                        s = s + v
                        sq = sq + v * v
                    t = (s * s - sq) * 0.5
                    z = jnp.sum(t)
                    zvec = jnp.where(lanes == b, jnp.full((_L,), z), zvec)
                plsc.store_scatter(zbuf, [jnp.full((_L,), c * _CB) + lanes], zvec)
                c2 = c + _RING

                @pl.when(c2 < _NCHUNK)
                def _():
                    fire(c2, rows_ref, sem)

            return carry

        lax.fori_loop(0, _NCHUNK // _RING, round_body, 0)

        for i in range(_BPW // _L):
            sl = pl.ds(i * _L, _L)
            v = zbuf[sl]
            zbuf[sl] = 1.0 / (1.0 + jnp.exp(-v))
        pltpu.sync_copy(zbuf, out_hbm.at[pl.ds(wid * _BPW, _BPW)])

    return fm_kernel(x_flat, off, table)
